# baseline (speedup 1.0000x reference)
import functools

import jax
import jax.numpy as jnp
from jax import lax
from jax.experimental import pallas as pl
from jax.experimental.pallas import tpu as pltpu
from jax.experimental.pallas import tpu_sc as plsc

N_NODES = 10000
N_EDGES = 320000
D_IN = 128
D_HID = 64
D_OUT = 16

NC, NS = 2, 16
NW = NC * NS
CHUNK = 128
EPW = N_EDGES // NW
NCH = 80
E_PAD = NW * NCH * CHUNK
KBUF = 5
SDEP = 2
N_PAD = 10240
RPT = N_PAD // NS
RCH = RPT // CHUNK
DUMMY = N_NODES
DEGW = 8


def _sc_mesh():
    return plsc.VectorSubcoreMesh(core_axis_name="c", subcore_axis_name="s",
                                  num_cores=NC, num_subcores=NS)


_SC_PARAMS = pltpu.CompilerParams(use_tc_tiling_on_sc=False)


def _deg_partials(edges):
    ones = jnp.ones((CHUNK, DEGW), jnp.float32)
    zeros = jnp.zeros((CHUNK, DEGW), jnp.float32)

    @functools.partial(
        pl.kernel,
        out_type=jax.ShapeDtypeStruct((NC, N_PAD, DEGW), jnp.float32),
        mesh=_sc_mesh(),
        scratch_types=[
            pltpu.VMEM((NCH, CHUNK), jnp.int32),
            pltpu.VMEM((CHUNK, DEGW), jnp.float32),
            pltpu.VMEM((CHUNK, DEGW), jnp.float32),
            pltpu.VMEM_SHARED((N_PAD, DEGW), jnp.float32),
            pltpu.SemaphoreType.DMA,
        ],
        compiler_params=_SC_PARAMS,
    )
    def degk(edges_hbm, ones_hbm, zeros_hbm, out_hbm, dstv, onesv, zbuf, acc,
             ssem):
        c = lax.axis_index("c")
        s = lax.axis_index("s")
        pltpu.sync_copy(edges_hbm.at[1, c * NS + s], dstv)
        pltpu.sync_copy(ones_hbm, onesv)
        pltpu.sync_copy(zeros_hbm, zbuf)
        base = s * RPT
        for t in range(RCH):
            pltpu.sync_copy(zbuf, acc.at[pl.ds(base + t * CHUNK, CHUNK)])
        plsc.subcore_barrier()

        pltpu.async_copy(onesv, acc.at[dstv.at[0]], ssem, add=True)

        def body(j, carry):
            pltpu.async_copy(onesv, acc.at[dstv.at[j + 1]], ssem, add=True)
            pltpu.make_async_copy(onesv, acc.at[dstv.at[j]], ssem).wait()
            return carry

        lax.fori_loop(0, NCH - 1, body, 0)
        pltpu.make_async_copy(onesv, acc.at[dstv.at[NCH - 1]], ssem).wait()
        plsc.subcore_barrier()
        for t in range(RCH):
            pltpu.sync_copy(acc.at[pl.ds(base + t * CHUNK, CHUNK)], zbuf)
            pltpu.sync_copy(zbuf, out_hbm.at[c, pl.ds(base + t * CHUNK, CHUNK)])

    return degk(edges, ones, zeros)


def _edge_aggregate(edges, y_pad, d):
    zeros = jnp.zeros((CHUNK, d), jnp.float32)

    @functools.partial(
        pl.kernel,
        out_type=jax.ShapeDtypeStruct((NC, N_PAD, d), jnp.float32),
        mesh=_sc_mesh(),
        scratch_types=[
            pltpu.VMEM((NCH, CHUNK), jnp.int32),
            pltpu.VMEM((NCH, CHUNK), jnp.int32),
            [pltpu.VMEM((CHUNK, d), jnp.float32) for _ in range(KBUF)],
            pltpu.VMEM((CHUNK, d), jnp.float32),
            pltpu.VMEM_SHARED((N_PAD, d), jnp.float32),
            [pltpu.SemaphoreType.DMA for _ in range(KBUF)],
            [pltpu.SemaphoreType.DMA for _ in range(KBUF)],
        ],
        compiler_params=_SC_PARAMS,
    )
    def sck(edges_hbm, y_hbm, zeros_hbm, out_hbm,
            srcv, dstv, bufs, zbuf, acc, gsems, ssems):
        c = lax.axis_index("c")
        s = lax.axis_index("s")
        w = c * NS + s
        pltpu.sync_copy(edges_hbm.at[0, w], srcv)
        pltpu.sync_copy(edges_hbm.at[1, w], dstv)
        pltpu.sync_copy(zeros_hbm, zbuf)
        base = s * RPT
        for t in range(RCH):
            pltpu.sync_copy(zbuf, acc.at[pl.ds(base + t * CHUNK, CHUNK)])
        plsc.subcore_barrier()

        for b in range(KBUF):
            pltpu.async_copy(y_hbm.at[srcv.at[b]], bufs[b], gsems[b])

        def body(t, carry):
            for b in range(KBUF):
                j = t * KBUF + b
                pltpu.make_async_copy(y_hbm.at[srcv.at[j]], bufs[b],
                                      gsems[b]).wait()
                pltpu.async_copy(bufs[b], acc.at[dstv.at[j]], ssems[b],
                                 add=True)
                jj = j - SDEP
                b2 = (b - SDEP) % KBUF

                @pl.when(jnp.logical_and(jj >= 0, jj + KBUF < NCH))
                def _():
                    pltpu.make_async_copy(bufs[b2], acc.at[dstv.at[jj]],
                                          ssems[b2]).wait()
                    pltpu.async_copy(y_hbm.at[srcv.at[jj + KBUF]], bufs[b2],
                                     gsems[b2])
            return carry

        lax.fori_loop(0, NCH // KBUF, body, 0)
        for b in range(KBUF):
            j = NCH - KBUF + b
            pltpu.make_async_copy(bufs[b], acc.at[dstv.at[j]],
                                  ssems[b]).wait()
        plsc.subcore_barrier()
        for t in range(RCH):
            pltpu.sync_copy(acc.at[pl.ds(base + t * CHUNK, CHUNK)], zbuf)
            pltpu.sync_copy(zbuf, out_hbm.at[c, pl.ds(base + t * CHUNK, CHUNK)])

    return sck(edges, y_pad, zeros)


_BR = 2000
_GRID = N_NODES // _BR


def _dspec():
    return pl.BlockSpec((2, _BR, 1), lambda i: (0, i, 0))


def _rspec(d):
    return pl.BlockSpec((_BR, d), lambda i: (i, 0))


def _aspec(d):
    return pl.BlockSpec((2, _BR, d), lambda i: (0, i, 0))


def _wspec(r, c):
    return pl.BlockSpec((r, c), lambda i: (0, 0))


def _dinv_of(d_ref):
    d = d_ref[...]
    return lax.rsqrt(d[0] + d[1] + 1.0)


def _tc_layer1(degs, x, W1):
    def body(d_ref, x_ref, w_ref, y_ref, s_ref):
        dinv = _dinv_of(d_ref)
        xw = jnp.dot(x_ref[...], w_ref[...], preferred_element_type=jnp.float32)
        y_ref[...] = dinv * xw
        s_ref[...] = (dinv * dinv) * xw

    return pl.pallas_call(
        body,
        grid=(_GRID,),
        in_specs=[_dspec(), _rspec(D_IN), _wspec(D_IN, D_HID)],
        out_specs=[_rspec(D_HID), _rspec(D_HID)],
        out_shape=[
            jax.ShapeDtypeStruct((N_PAD, D_HID), jnp.float32),
            jax.ShapeDtypeStruct((N_NODES, D_HID), jnp.float32),
        ],
    )(degs, x, W1)


def _tc_layer2(degs, acc1, self1, b1, W2):
    def body(d_ref, a_ref, s1_ref, b1_ref, w2_ref, y_ref, s_ref):
        dinv = _dinv_of(d_ref)
        a = a_ref[...]
        h = jnp.maximum(dinv * (a[0] + a[1]) + s1_ref[...] + b1_ref[...], 0.0)
        hw = jnp.dot(h, w2_ref[...], preferred_element_type=jnp.float32)
        y2 = dinv * hw
        y_ref[...] = y2
        s_ref[...] = dinv * y2

    return pl.pallas_call(
        body,
        grid=(_GRID,),
        in_specs=[_dspec(), _aspec(D_HID), _rspec(D_HID),
                  _wspec(1, D_HID), _wspec(D_HID, D_OUT)],
        out_specs=[_rspec(D_OUT), _rspec(D_OUT)],
        out_shape=[
            jax.ShapeDtypeStruct((N_PAD, D_OUT), jnp.float32),
            jax.ShapeDtypeStruct((N_NODES, D_OUT), jnp.float32),
        ],
    )(degs, acc1, self1, b1, W2)


def _tc_layer3(degs, acc2, self2, b2):
    def body(d_ref, a_ref, s2_ref, b2_ref, o_ref):
        dinv = _dinv_of(d_ref)
        a = a_ref[...]
        o_ref[...] = dinv * (a[0] + a[1]) + s2_ref[...] + b2_ref[...]

    return pl.pallas_call(
        body,
        grid=(_GRID,),
        in_specs=[_dspec(), _aspec(D_OUT), _rspec(D_OUT), _wspec(1, D_OUT)],
        out_specs=pl.BlockSpec((_BR, D_OUT), lambda i: (i, 0)),
        out_shape=jax.ShapeDtypeStruct((N_NODES, D_OUT), jnp.float32),
    )(degs, acc2, self2, b2)


def kernel(x, edge_index, W1, b1, W2, b2):
    ei = edge_index.astype(jnp.int32)
    pad = DUMMY + jnp.arange(E_PAD - N_EDGES, dtype=jnp.int32) % (N_PAD - DUMMY)
    pad2 = jnp.broadcast_to(pad, (2, E_PAD - N_EDGES))
    edges = jnp.concatenate([ei, pad2], axis=1).reshape(2, NW, NCH, CHUNK)

    degp = _deg_partials(edges)
    degs = degp[:, :, :1]
    y1, self1 = _tc_layer1(degs, x, W1)
    acc1 = _edge_aggregate(edges, y1, D_HID)
    y2, self2 = _tc_layer2(degs, acc1, self1, b1.reshape(1, D_HID), W2)
    acc2 = _edge_aggregate(edges, y2, D_OUT)
    return _tc_layer3(degs, acc2, self2, b2.reshape(1, D_OUT))


# speedup vs baseline: 1.0237x; 1.0237x over previous
import functools

import jax
import jax.numpy as jnp
from jax import lax
from jax.experimental import pallas as pl
from jax.experimental.pallas import tpu as pltpu
from jax.experimental.pallas import tpu_sc as plsc

N_NODES = 10000
N_EDGES = 320000
D_IN = 128
D_HID = 64
D_OUT = 16

NC, NS = 2, 16
NW = NC * NS
CHUNK = 128
EPW = N_EDGES // NW
NCH = 80
E_PAD = NW * NCH * CHUNK
KBUF = 4
N_PAD = 10240
RPT = N_PAD // NS
RCH = RPT // CHUNK
DUMMY = N_NODES
DEGW = 8


def _sc_mesh():
    return plsc.VectorSubcoreMesh(core_axis_name="c", subcore_axis_name="s",
                                  num_cores=NC, num_subcores=NS)


_SC_PARAMS = pltpu.CompilerParams(use_tc_tiling_on_sc=False)


def _deg_partials(edges):
    ones = jnp.ones((CHUNK, DEGW), jnp.float32)
    zeros = jnp.zeros((CHUNK, DEGW), jnp.float32)

    @functools.partial(
        pl.kernel,
        out_type=jax.ShapeDtypeStruct((NC, N_PAD, DEGW), jnp.float32),
        mesh=_sc_mesh(),
        scratch_types=[
            pltpu.VMEM((NCH, CHUNK), jnp.int32),
            pltpu.VMEM((CHUNK, DEGW), jnp.float32),
            pltpu.VMEM((CHUNK, DEGW), jnp.float32),
            pltpu.VMEM_SHARED((N_PAD, DEGW), jnp.float32),
            pltpu.SemaphoreType.DMA,
        ],
        compiler_params=_SC_PARAMS,
    )
    def degk(edges_hbm, ones_hbm, zeros_hbm, out_hbm, dstv, onesv, zbuf, acc,
             ssem):
        c = lax.axis_index("c")
        s = lax.axis_index("s")
        pltpu.sync_copy(edges_hbm.at[1, c * NS + s], dstv)
        pltpu.sync_copy(ones_hbm, onesv)
        pltpu.sync_copy(zeros_hbm, zbuf)
        base = s * RPT
        for t in range(RCH):
            pltpu.sync_copy(zbuf, acc.at[pl.ds(base + t * CHUNK, CHUNK)])
        plsc.subcore_barrier()

        pltpu.async_copy(onesv, acc.at[dstv.at[0]], ssem, add=True)

        def body(j, carry):
            pltpu.async_copy(onesv, acc.at[dstv.at[j + 1]], ssem, add=True)
            pltpu.make_async_copy(onesv, acc.at[dstv.at[j]], ssem).wait()
            return carry

        lax.fori_loop(0, NCH - 1, body, 0)
        pltpu.make_async_copy(onesv, acc.at[dstv.at[NCH - 1]], ssem).wait()
        plsc.subcore_barrier()
        for t in range(RCH):
            pltpu.sync_copy(acc.at[pl.ds(base + t * CHUNK, CHUNK)], zbuf)
            pltpu.sync_copy(zbuf, out_hbm.at[c, pl.ds(base + t * CHUNK, CHUNK)])

    return degk(edges, ones, zeros)


def _edge_aggregate(edges, y_pad, d):
    zeros = jnp.zeros((CHUNK, d), jnp.float32)

    @functools.partial(
        pl.kernel,
        out_type=jax.ShapeDtypeStruct((NC, N_PAD, d), jnp.float32),
        mesh=_sc_mesh(),
        scratch_types=[
            pltpu.VMEM((NCH, CHUNK), jnp.int32),
            pltpu.VMEM((NCH, CHUNK), jnp.int32),
            [pltpu.VMEM((CHUNK, d), jnp.float32) for _ in range(KBUF)],
            pltpu.VMEM((CHUNK, d), jnp.float32),
            pltpu.VMEM_SHARED((N_PAD, d), jnp.float32),
            [pltpu.SemaphoreType.DMA for _ in range(KBUF)],
        ],
        compiler_params=_SC_PARAMS,
    )
    def sck(edges_hbm, y_hbm, zeros_hbm, out_hbm,
            srcv, dstv, bufs, zbuf, acc, gsems):
        c = lax.axis_index("c")
        s = lax.axis_index("s")
        w = c * NS + s
        pltpu.sync_copy(edges_hbm.at[0, w], srcv)
        pltpu.sync_copy(edges_hbm.at[1, w], dstv)
        pltpu.sync_copy(zeros_hbm, zbuf)
        base = s * RPT
        for t in range(RCH):
            pltpu.sync_copy(zbuf, acc.at[pl.ds(base + t * CHUNK, CHUNK)])
        plsc.subcore_barrier()

        for b in range(KBUF):
            pltpu.async_copy(y_hbm.at[srcv.at[b]], bufs[b], gsems[b])

        def body(t, carry):
            for b in range(KBUF):
                j = t * KBUF + b
                pltpu.make_async_copy(y_hbm.at[srcv.at[j]], bufs[b],
                                      gsems[b]).wait()
                pltpu.sync_copy(bufs[b], acc.at[dstv.at[j]], add=True)

                @pl.when(j + KBUF < NCH)
                def _():
                    pltpu.async_copy(y_hbm.at[srcv.at[j + KBUF]], bufs[b],
                                     gsems[b])
            return carry

        lax.fori_loop(0, NCH // KBUF, body, 0)
        plsc.subcore_barrier()
        for t in range(RCH):
            pltpu.sync_copy(acc.at[pl.ds(base + t * CHUNK, CHUNK)], zbuf)
            pltpu.sync_copy(zbuf, out_hbm.at[c, pl.ds(base + t * CHUNK, CHUNK)])

    return sck(edges, y_pad, zeros)


def _dinv_of(d_ref):
    d = d_ref[...]
    return lax.rsqrt(d[0, :N_NODES] + d[1, :N_NODES] + 1.0)


def _tc_matmul1(x, W1):
    def body(x_ref, w_ref, o_ref):
        o_ref[...] = jnp.dot(x_ref[...], w_ref[...],
                             preferred_element_type=jnp.float32)

    return pl.pallas_call(
        body,
        out_shape=jax.ShapeDtypeStruct((N_NODES, D_HID), jnp.float32),
    )(x, W1)


def _tc_layer1(degs, xw, ):
    def body(d_ref, xw_ref, y_ref, s_ref):
        dinv = _dinv_of(d_ref)
        xwv = xw_ref[...]
        y_ref[:N_NODES, :] = dinv * xwv
        s_ref[...] = (dinv * dinv) * xwv

    return pl.pallas_call(
        body,
        out_shape=[
            jax.ShapeDtypeStruct((N_PAD, D_HID), jnp.float32),
            jax.ShapeDtypeStruct((N_NODES, D_HID), jnp.float32),
        ],
    )(degs, xw)


def _tc_layer2(degs, acc1, self1, b1, W2):
    def body(d_ref, a_ref, s1_ref, b1_ref, w2_ref, y_ref, s_ref):
        dinv = _dinv_of(d_ref)
        a = a_ref[...]
        h = jnp.maximum(
            dinv * (a[0, :N_NODES] + a[1, :N_NODES]) + s1_ref[...]
            + b1_ref[...], 0.0)
        hw = jnp.dot(h, w2_ref[...], preferred_element_type=jnp.float32)
        y2 = dinv * hw
        y_ref[:N_NODES, :] = y2
        s_ref[...] = dinv * y2

    return pl.pallas_call(
        body,
        out_shape=[
            jax.ShapeDtypeStruct((N_PAD, D_OUT), jnp.float32),
            jax.ShapeDtypeStruct((N_NODES, D_OUT), jnp.float32),
        ],
    )(degs, acc1, self1, b1, W2)


def _tc_layer3(degs, acc2, self2, b2):
    def body(d_ref, a_ref, s2_ref, b2_ref, o_ref):
        dinv = _dinv_of(d_ref)
        a = a_ref[...]
        o_ref[...] = (dinv * (a[0, :N_NODES] + a[1, :N_NODES])
                      + s2_ref[...] + b2_ref[...])

    return pl.pallas_call(
        body,
        out_shape=jax.ShapeDtypeStruct((N_NODES, D_OUT), jnp.float32),
    )(degs, acc2, self2, b2)


def kernel(x, edge_index, W1, b1, W2, b2):
    ei = edge_index.astype(jnp.int32)
    pad = DUMMY + jnp.arange(E_PAD - N_EDGES, dtype=jnp.int32) % (N_PAD - DUMMY)
    pad2 = jnp.broadcast_to(pad, (2, E_PAD - N_EDGES))
    edges = jnp.concatenate([ei, pad2], axis=1).reshape(2, NW, NCH, CHUNK)

    degp = _deg_partials(edges)
    degs = degp[:, :, :1]
    xw = _tc_matmul1(x, W1)
    y1, self1 = _tc_layer1(degs, xw)
    acc1 = _edge_aggregate(edges, y1, D_HID)
    y2, self2 = _tc_layer2(degs, acc1, self1, b1.reshape(1, D_HID), W2)
    acc2 = _edge_aggregate(edges, y2, D_OUT)
    return _tc_layer3(degs, acc2, self2, b2.reshape(1, D_OUT))
